# baseline (speedup 1.0000x reference)
import functools
import math

import jax
import jax.numpy as jnp
from jax import lax
from jax.experimental import pallas as pl
from jax.experimental.pallas import tpu as pltpu
from jax.experimental.pallas import tpu_sc as plsc

_ALPHA = 0.2
_EPS = 1e-5
_B = 16384
_D = 64
_T = 200
_BLK = 2048
_NB = _B // _BLK
_PR = _BLK // 2
_OR = _BLK // 128
_CB = 2048
_NPB = _B // _CB
_CH = 128
_L2A = math.log2(1.0 - _ALPHA)


_SMALL_ROWS = 2216
_CAT_OFF = 208
_STORE_OFF = 1216


def _sc_gather_small(time_i, rel, cat_off, store_off, small_tab):
  info = plsc.get_sparse_core_info()
  nc, ns = info.num_cores, info.num_subcores
  nw = nc * ns
  bpw = _B // nw
  nch = bpw // _CH
  mesh = plsc.VectorSubcoreMesh(core_axis_name="c", subcore_axis_name="s")
  out_type = tuple(
      jax.ShapeDtypeStruct((_B, _D), jnp.float32) for _ in range(4))

  @functools.partial(
      pl.kernel,
      mesh=mesh,
      out_type=out_type,
      compiler_params=pltpu.CompilerParams(use_tc_tiling_on_sc=False),
      scratch_types=[
          pltpu.VMEM((4, bpw), jnp.int32),
          pltpu.VMEM((bpw, _D), jnp.float32),
          pltpu.VMEM((bpw, _D), jnp.float32),
          pltpu.VMEM_SHARED((_SMALL_ROWS, _D), jnp.float32),
          pltpu.SemaphoreType.DMA,
          pltpu.SemaphoreType.DMA,
      ],
  )
  def gather_kernel(i0_h, i1_h, i2_h, i3_h, tab_h,
                    o0, o1, o2, o3,
                    idx_v, rows_a, rows_b, tab_s, sem_a, sem_b):
    wid = lax.axis_index("s") * nc + lax.axis_index("c")
    base = wid * bpw
    idxs = (i0_h, i1_h, i2_h, i3_h)
    outs = (o0, o1, o2, o3)
    bufs = (rows_a, rows_b)
    sems = (sem_a, sem_b)

    @pl.when(lax.axis_index("s") == 0)
    def _stage():
      pltpu.sync_copy(tab_h, tab_s)

    for t in range(4):
      pltpu.sync_copy(idxs[t].at[pl.ds(base, bpw)], idx_v.at[t])
    plsc.subcore_barrier()

    def fire(t):
      buf = bufs[t % 2]
      sem = sems[t % 2]
      handles = []
      for j in range(nch):
        handles.append(
            pltpu.async_copy(tab_s.at[idx_v.at[t, pl.ds(j * _CH, _CH)]],
                             buf.at[pl.ds(j * _CH, _CH)], sem))
      return handles

    pending = fire(0)
    for t in range(4):
      for h in pending:
        h.wait()
      cur = bufs[t % 2]
      if t + 1 < 4:
        pending = fire(t + 1)
      pltpu.sync_copy(cur, outs[t].at[pl.ds(base, bpw)])

  return gather_kernel(time_i, rel, cat_off, store_off, small_tab)


def _sc_gather_item(item, item_table):
  info = plsc.get_sparse_core_info()
  nc, ns = info.num_cores, info.num_subcores
  nw = nc * ns
  bpw = _B // nw
  nch = bpw // _CH
  mesh = plsc.VectorSubcoreMesh(core_axis_name="c", subcore_axis_name="s")

  @functools.partial(
      pl.kernel,
      mesh=mesh,
      out_type=jax.ShapeDtypeStruct((_B, _D), jnp.float32),
      compiler_params=pltpu.CompilerParams(use_tc_tiling_on_sc=False),
      scratch_types=[
          pltpu.VMEM((bpw,), jnp.int32),
          pltpu.VMEM((bpw, _D), jnp.float32),
          pltpu.SemaphoreType.DMA,
      ],
  )
  def gather_kernel(idx_h, tab_h, out_h, idx_v, rows_v, sem):
    wid = lax.axis_index("s") * nc + lax.axis_index("c")
    base = wid * bpw
    pltpu.sync_copy(idx_h.at[pl.ds(base, bpw)], idx_v)
    handles = []
    for j in range(nch):
      handles.append(
          pltpu.async_copy(tab_h.at[idx_v.at[pl.ds(j * _CH, _CH)]],
                           rows_v.at[pl.ds(j * _CH, _CH)], sem))
    for h in handles:
      h.wait()
    pltpu.sync_copy(rows_v, out_h.at[pl.ds(base, bpw)])

  return gather_kernel(item, item_table)


_TCOLS = 4096


def _item_repack(item_t):
  n = item_t.shape[1]
  nblk = (n + _TCOLS - 1) // _TCOLS

  def body(src_ref, out_ref):
    s = src_ref[...]
    col = (lax.broadcasted_iota(jnp.int32, (_D, _TCOLS), 1)
           + pl.program_id(0) * _TCOLS)
    s = jnp.where(col < n, s, 0.0)
    lane = lax.broadcasted_iota(jnp.int32, (512, 1024), 1)
    sub = lax.broadcasted_iota(jnp.int32, (512, 1024), 0)
    se = (lane == 2 * sub).astype(jnp.float32)
    so = (lane == 2 * sub + 1).astype(jnp.float32)
    dn = (((1,), (1,)), ((), ()))
    chunks = []
    for q in range(_TCOLS // 1024):
      sq = s[:, q * 1024:(q + 1) * 1024]
      left = lax.dot_general(se, sq, dn)
      right = lax.dot_general(so, sq, dn)
      chunks.append(jnp.concatenate([left, right], axis=1))
    out_ref[...] = jnp.concatenate(chunks, axis=0)

  out = pl.pallas_call(
      body,
      grid=(nblk,),
      in_specs=[pl.BlockSpec((_D, _TCOLS), lambda b: (0, b))],
      out_specs=pl.BlockSpec((_TCOLS // 2, 128), lambda b: (b, 0)),
      out_shape=jax.ShapeDtypeStruct((nblk * _TCOLS // 2, 128), jnp.float32),
  )(item_t)
  return out.reshape(nblk * _TCOLS, _D)


def _pop_body(ph_ref, tf_ref, out_ref):
  ph = ph_ref[...]
  tb = tf_ref[...].reshape(1, _CB)
  tb = jnp.maximum(tb - 1.0, 0.0)
  k = lax.broadcasted_iota(jnp.int32, (_T, 1), 0).astype(jnp.float32)
  col = jnp.exp2(tb * _L2A)
  row = jnp.exp2(k * (-_L2A))
  w = col * row
  coef = jnp.where(k > tb, 0.0, jnp.where(k == 0.0, w, _ALPHA * w))
  pop = jnp.sum(coef * ph, axis=0, keepdims=True)
  out_ref[...] = pop.reshape(1, 1, _CB)


def _fold(v):
  return v[:, 0:_D] + v[:, _D:2 * _D]


def _stats_body(ie_ref, te_ref, re_ref, ce_ref, se_ref, out_ref, acc_ref):
  b = pl.program_id(0)

  @pl.when(b == 0)
  def _init():
    acc_ref[...] = jnp.zeros_like(acc_ref)

  ie = ie_ref[...]
  te = te_ref[...]
  re = re_ref[...]
  ce = ce_ref[...]
  se = se_ref[...]
  gap = re - te
  cols = (ie, te, re, ce, se)
  sums = [_fold(jnp.sum(x, axis=0, keepdims=True)) for x in cols]
  sqs = [_fold(jnp.sum(x * x, axis=0, keepdims=True)) for x in cols]
  gsum = _fold(jnp.sum(gap, axis=0, keepdims=True))
  gsq = _fold(jnp.sum(gap * gap, axis=0, keepdims=True))
  z = jnp.zeros((4, _D), jnp.float32)
  acc_ref[...] += jnp.concatenate(sums + sqs + [gsum, gsq, z], axis=0)

  @pl.when(b == _NB - 1)
  def _emit():
    out_ref[...] = acc_ref[...]


def _final_body(st_ref, pop_ref, ie_ref, te_ref, re_ref, ce_ref, se_ref,
                gt_ref, bt_ref, wt_ref, bt0_ref,
                gs_ref, bs_ref, ws_ref, bs0_ref, aw_ref,
                opop_ref, otime_ref, oside_ref, oout_ref):
  st = st_ref[...]
  inv_n = 1.0 / _B
  mu = st[0:5] * inv_n
  ex2 = st[5:10] * inv_n
  mu_g = st[10:11] * inv_n
  ex2_g = st[11:12] * inv_n
  var = ex2 - mu * mu
  var_g = ex2_g - mu_g * mu_g
  mu_t = jnp.concatenate([mu_g, mu[0:1], mu[1:2], mu[2:3]], axis=0)
  var_t = jnp.concatenate([var_g, var[0:1], var[1:2], var[2:3]], axis=0)
  w4 = wt_ref[...]
  v4 = gt_ref[...] * w4 * lax.rsqrt(var_t + _EPS)
  c_t = jnp.sum(bt_ref[...] * w4) - jnp.sum(mu_t * v4) + bt0_ref[...]
  w2 = ws_ref[...]
  v2 = gs_ref[...] * w2 * lax.rsqrt(var[3:5] + _EPS)
  c_s = jnp.sum(bs_ref[...] * w2) - jnp.sum(mu[3:5] * v2) + bs0_ref[...]
  aw = aw_ref[...]
  e = jnp.exp(aw - jnp.max(aw))
  wsm = e / jnp.sum(e)
  wa = wsm[0:1, 0:1]
  wb = wsm[0:1, 1:2]
  wc = wsm[0:1, 2:3]

  vv4 = jnp.concatenate([v4, v4], axis=1)
  vv2 = jnp.concatenate([v2, v2], axis=1)
  ie = ie_ref[...]
  te = te_ref[...]
  re = re_ref[...]
  gap = re - te
  yt = (gap * vv4[0:1] + ie * vv4[1:2] + te * vv4[2:3] + re * vv4[3:4])
  ys = ce_ref[...] * vv2[0:1] + se_ref[...] * vv2[1:2]
  dte = jnp.sum(yt[:, 0:_D], axis=1, keepdims=True)
  dto = jnp.sum(yt[:, _D:], axis=1, keepdims=True)
  dse = jnp.sum(ys[:, 0:_D], axis=1, keepdims=True)
  dso = jnp.sum(ys[:, _D:], axis=1, keepdims=True)

  lane = lax.broadcasted_iota(jnp.int32, (_D, 128), 1)
  sub = lax.broadcasted_iota(jnp.int32, (_D, 128), 0)
  pe = (lane == 2 * sub).astype(jnp.float32)
  po = (lane == 2 * sub + 1).astype(jnp.float32)
  dn = (((1,), (0,)), ((), ()))

  def interleave(ev, od):
    evq = ev.reshape(_OR, _D)
    odq = od.reshape(_OR, _D)
    return lax.dot_general(evq, pe, dn) + lax.dot_general(odq, po, dn)

  dt = interleave(dte, dto)
  ds = interleave(dse, dso)

  t_out = jnp.maximum(dt + c_t, 0.0)
  s_out = ds + c_s
  w_pop = pop_ref[...] * wa
  w_time = t_out * wb
  w_side = s_out * wc
  opop_ref[...] = w_pop
  otime_ref[...] = w_time
  oside_ref[...] = w_side
  oout_ref[...] = w_pop + w_time + w_side


def kernel(pop_history, item, time, release_time, category, store,
           item_table, time_table, cat_table, store_table,
           gamma_time, beta_time, W_time, b_time,
           gamma_side, beta_side, W_side, b_side, attn_w):
  i32 = jnp.int32
  z7 = jnp.zeros((7, _D), jnp.float32)
  small_tab = jnp.concatenate([time_table, z7, cat_table, z7, store_table],
                              axis=0)
  time_e, rel_e, cat_e, store_e = _sc_gather_small(
      time.astype(i32), release_time.astype(i32),
      category.astype(i32) + _CAT_OFF, store.astype(i32) + _STORE_OFF,
      small_tab)
  item_e = _sc_gather_item(item.astype(i32), _item_repack(item_table.T))
  packed = [x.reshape(_B // 2, 128)
            for x in (item_e, time_e, rel_e, cat_e, store_e)]

  tf = time.astype(jnp.float32).reshape(_NPB, 1, _CB)
  pop3 = pl.pallas_call(
      _pop_body,
      grid=(_NPB,),
      in_specs=[
          pl.BlockSpec((_T, _CB), lambda b: (0, b)),
          pl.BlockSpec((1, 1, _CB), lambda b: (b, 0, 0)),
      ],
      out_specs=pl.BlockSpec((1, 1, _CB), lambda b: (b, 0, 0)),
      out_shape=jax.ShapeDtypeStruct((_NPB, 1, _CB), jnp.float32),
  )(pop_history.T, tf)
  pop_pk = pop3.reshape(128, 128)

  blk = lambda b: (b, 0)
  full = lambda b: (0, 0)
  espec = pl.BlockSpec((_PR, 128), blk)
  stats = pl.pallas_call(
      _stats_body,
      grid=(_NB,),
      in_specs=[espec] * 5,
      out_specs=pl.BlockSpec((16, _D), full),
      out_shape=jax.ShapeDtypeStruct((16, _D), jnp.float32),
      scratch_shapes=[pltpu.VMEM((16, _D), jnp.float32)],
  )(*packed)

  gt = gamma_time.reshape(4, _D)
  bt = beta_time.reshape(4, _D)
  wt = W_time.reshape(4, _D)
  bt0 = b_time.reshape(1, 1)
  gs = gamma_side.reshape(2, _D)
  bs = beta_side.reshape(2, _D)
  ws = W_side.reshape(2, _D)
  bs0 = b_side.reshape(1, 1)
  aw = attn_w.reshape(1, 3)

  out_spec = pl.BlockSpec((_OR, 128), blk)
  w_pop, w_time, w_side, output = pl.pallas_call(
      _final_body,
      grid=(_NB,),
      in_specs=[
          pl.BlockSpec((16, _D), full),
          pl.BlockSpec((_OR, 128), blk),
          espec, espec, espec, espec, espec,
          pl.BlockSpec((4, _D), full),
          pl.BlockSpec((4, _D), full),
          pl.BlockSpec((4, _D), full),
          pl.BlockSpec((1, 1), full),
          pl.BlockSpec((2, _D), full),
          pl.BlockSpec((2, _D), full),
          pl.BlockSpec((2, _D), full),
          pl.BlockSpec((1, 1), full),
          pl.BlockSpec((1, 3), full),
      ],
      out_specs=[out_spec] * 4,
      out_shape=[jax.ShapeDtypeStruct((128, 128), jnp.float32)] * 4,
  )(stats, pop_pk, *packed,
    gt, bt, wt, bt0, gs, bs, ws, bs0, aw)

  return tuple(o.reshape(_B, 1) for o in (w_pop, w_time, w_side, output))


# speedup vs baseline: 1.2941x; 1.2941x over previous
import functools
import math

import jax
import jax.numpy as jnp
from jax import lax
from jax.experimental import pallas as pl
from jax.experimental.pallas import tpu as pltpu
from jax.experimental.pallas import tpu_sc as plsc

_ALPHA = 0.2
_EPS = 1e-5
_B = 16384
_D = 64
_T = 200
_BLK = 2048
_NB = _B // _BLK
_PR = _BLK // 2
_OR = _BLK // 128
_CB = 2048
_NPB = _B // _CB
_CH = 128
_L2A = math.log2(1.0 - _ALPHA)


_SMALL_ROWS = 2216
_CAT_OFF = 208
_STORE_OFF = 1216


def _sc_gather_small(time_i, rel, cat_off, store_off, small_tab):
  info = plsc.get_sparse_core_info()
  nc, ns = info.num_cores, info.num_subcores
  nw = nc * ns
  bpw = _B // nw
  nch = bpw // _CH
  mesh = plsc.VectorSubcoreMesh(core_axis_name="c", subcore_axis_name="s")
  out_type = tuple(
      jax.ShapeDtypeStruct((_B, _D), jnp.float32) for _ in range(4))

  @functools.partial(
      pl.kernel,
      mesh=mesh,
      out_type=out_type,
      compiler_params=pltpu.CompilerParams(use_tc_tiling_on_sc=False),
      scratch_types=[
          pltpu.VMEM((4, bpw), jnp.int32),
          pltpu.VMEM((bpw, _D), jnp.float32),
          pltpu.VMEM((bpw, _D), jnp.float32),
          pltpu.VMEM_SHARED((_SMALL_ROWS, _D), jnp.float32),
          pltpu.SemaphoreType.DMA,
          pltpu.SemaphoreType.DMA,
      ],
  )
  def gather_kernel(i0_h, i1_h, i2_h, i3_h, tab_h,
                    o0, o1, o2, o3,
                    idx_v, rows_a, rows_b, tab_s, sem_a, sem_b):
    wid = lax.axis_index("s") * nc + lax.axis_index("c")
    base = wid * bpw
    idxs = (i0_h, i1_h, i2_h, i3_h)
    outs = (o0, o1, o2, o3)
    bufs = (rows_a, rows_b)
    sems = (sem_a, sem_b)

    @pl.when(lax.axis_index("s") == 0)
    def _stage():
      pltpu.sync_copy(tab_h, tab_s)

    for t in range(4):
      pltpu.sync_copy(idxs[t].at[pl.ds(base, bpw)], idx_v.at[t])
    plsc.subcore_barrier()

    def fire(t):
      buf = bufs[t % 2]
      sem = sems[t % 2]
      handles = []
      for j in range(nch):
        handles.append(
            pltpu.async_copy(tab_s.at[idx_v.at[t, pl.ds(j * _CH, _CH)]],
                             buf.at[pl.ds(j * _CH, _CH)], sem))
      return handles

    pending = fire(0)
    for t in range(4):
      for h in pending:
        h.wait()
      cur = bufs[t % 2]
      if t + 1 < 4:
        pending = fire(t + 1)
      pltpu.sync_copy(cur, outs[t].at[pl.ds(base, bpw)])

  return gather_kernel(time_i, rel, cat_off, store_off, small_tab)


def _sc_gather_item(item, item_table):
  info = plsc.get_sparse_core_info()
  nc, ns = info.num_cores, info.num_subcores
  nw = nc * ns
  bpw = _B // nw
  nch = bpw // _CH
  mesh = plsc.VectorSubcoreMesh(core_axis_name="c", subcore_axis_name="s")

  @functools.partial(
      pl.kernel,
      mesh=mesh,
      out_type=jax.ShapeDtypeStruct((_B, _D), jnp.float32),
      compiler_params=pltpu.CompilerParams(use_tc_tiling_on_sc=False),
      scratch_types=[
          pltpu.VMEM((bpw,), jnp.int32),
          pltpu.VMEM((bpw, _D), jnp.float32),
          pltpu.SemaphoreType.DMA,
      ],
  )
  def gather_kernel(idx_h, tab_h, out_h, idx_v, rows_v, sem):
    wid = lax.axis_index("s") * nc + lax.axis_index("c")
    base = wid * bpw
    pltpu.sync_copy(idx_h.at[pl.ds(base, bpw)], idx_v)
    handles = []
    for j in range(nch):
      handles.append(
          pltpu.async_copy(tab_h.at[idx_v.at[pl.ds(j * _CH, _CH)]],
                           rows_v.at[pl.ds(j * _CH, _CH)], sem))
    for h in handles:
      h.wait()
    pltpu.sync_copy(rows_v, out_h.at[pl.ds(base, bpw)])

  return gather_kernel(item, item_table)


_TCOLS = 4096


def _item_repack(item_t):
  n = item_t.shape[1]
  nblk = (n + _TCOLS - 1) // _TCOLS

  def body(src_ref, out_ref):
    s = src_ref[...]
    col = (lax.broadcasted_iota(jnp.int32, (_D, _TCOLS), 1)
           + pl.program_id(0) * _TCOLS)
    s = jnp.where(col < n, s, 0.0)
    lane = lax.broadcasted_iota(jnp.int32, (_D, 128), 1)
    sub = lax.broadcasted_iota(jnp.int32, (_D, 128), 0)
    se = (lane == 2 * sub).astype(jnp.float32)
    so = (lane == 2 * sub + 1).astype(jnp.float32)
    dn = (((1,), (1,)), ((), ()))
    chunks = []
    for q in range(_TCOLS // 128):
      sq = s[:, q * 128:(q + 1) * 128]
      left = lax.dot_general(se, sq, dn)
      right = lax.dot_general(so, sq, dn)
      chunks.append(jnp.concatenate([left, right], axis=1))
    out_ref[...] = jnp.concatenate(chunks, axis=0)

  out = pl.pallas_call(
      body,
      grid=(nblk,),
      in_specs=[pl.BlockSpec((_D, _TCOLS), lambda b: (0, b))],
      out_specs=pl.BlockSpec((_TCOLS // 2, 128), lambda b: (b, 0)),
      out_shape=jax.ShapeDtypeStruct((nblk * _TCOLS // 2, 128), jnp.float32),
  )(item_t)
  return out.reshape(nblk * _TCOLS, _D)


def _pop_body(ph_ref, tf_ref, out_ref):
  ph = ph_ref[...]
  tb = tf_ref[...].reshape(1, _CB)
  tb = jnp.maximum(tb - 1.0, 0.0)
  k = lax.broadcasted_iota(jnp.int32, (_T, 1), 0).astype(jnp.float32)
  col = jnp.exp2(tb * _L2A)
  row = jnp.exp2(k * (-_L2A))
  w = col * row
  coef = jnp.where(k > tb, 0.0, jnp.where(k == 0.0, w, _ALPHA * w))
  pop = jnp.sum(coef * ph, axis=0, keepdims=True)
  out_ref[...] = pop.reshape(1, 1, _CB)


def _fold(v):
  return v[:, 0:_D] + v[:, _D:2 * _D]


def _stats_body(ie_ref, te_ref, re_ref, ce_ref, se_ref, out_ref, acc_ref):
  b = pl.program_id(0)

  @pl.when(b == 0)
  def _init():
    acc_ref[...] = jnp.zeros_like(acc_ref)

  ie = ie_ref[...]
  te = te_ref[...]
  re = re_ref[...]
  ce = ce_ref[...]
  se = se_ref[...]
  gap = re - te
  cols = (ie, te, re, ce, se)
  sums = [_fold(jnp.sum(x, axis=0, keepdims=True)) for x in cols]
  sqs = [_fold(jnp.sum(x * x, axis=0, keepdims=True)) for x in cols]
  gsum = _fold(jnp.sum(gap, axis=0, keepdims=True))
  gsq = _fold(jnp.sum(gap * gap, axis=0, keepdims=True))
  z = jnp.zeros((4, _D), jnp.float32)
  acc_ref[...] += jnp.concatenate(sums + sqs + [gsum, gsq, z], axis=0)

  @pl.when(b == _NB - 1)
  def _emit():
    out_ref[...] = acc_ref[...]


def _final_body(st_ref, pop_ref, ie_ref, te_ref, re_ref, ce_ref, se_ref,
                gt_ref, bt_ref, wt_ref, bt0_ref,
                gs_ref, bs_ref, ws_ref, bs0_ref, aw_ref,
                opop_ref, otime_ref, oside_ref, oout_ref):
  st = st_ref[...]
  inv_n = 1.0 / _B
  mu = st[0:5] * inv_n
  ex2 = st[5:10] * inv_n
  mu_g = st[10:11] * inv_n
  ex2_g = st[11:12] * inv_n
  var = ex2 - mu * mu
  var_g = ex2_g - mu_g * mu_g
  mu_t = jnp.concatenate([mu_g, mu[0:1], mu[1:2], mu[2:3]], axis=0)
  var_t = jnp.concatenate([var_g, var[0:1], var[1:2], var[2:3]], axis=0)
  w4 = wt_ref[...]
  v4 = gt_ref[...] * w4 * lax.rsqrt(var_t + _EPS)
  c_t = jnp.sum(bt_ref[...] * w4) - jnp.sum(mu_t * v4) + bt0_ref[...]
  w2 = ws_ref[...]
  v2 = gs_ref[...] * w2 * lax.rsqrt(var[3:5] + _EPS)
  c_s = jnp.sum(bs_ref[...] * w2) - jnp.sum(mu[3:5] * v2) + bs0_ref[...]
  aw = aw_ref[...]
  e = jnp.exp(aw - jnp.max(aw))
  wsm = e / jnp.sum(e)
  wa = wsm[0:1, 0:1]
  wb = wsm[0:1, 1:2]
  wc = wsm[0:1, 2:3]

  vv4 = jnp.concatenate([v4, v4], axis=1)
  vv2 = jnp.concatenate([v2, v2], axis=1)
  ie = ie_ref[...]
  te = te_ref[...]
  re = re_ref[...]
  gap = re - te
  yt = (gap * vv4[0:1] + ie * vv4[1:2] + te * vv4[2:3] + re * vv4[3:4])
  ys = ce_ref[...] * vv2[0:1] + se_ref[...] * vv2[1:2]
  dte = jnp.sum(yt[:, 0:_D], axis=1, keepdims=True)
  dto = jnp.sum(yt[:, _D:], axis=1, keepdims=True)
  dse = jnp.sum(ys[:, 0:_D], axis=1, keepdims=True)
  dso = jnp.sum(ys[:, _D:], axis=1, keepdims=True)

  lane = lax.broadcasted_iota(jnp.int32, (_D, 128), 1)
  sub = lax.broadcasted_iota(jnp.int32, (_D, 128), 0)
  pe = (lane == 2 * sub).astype(jnp.float32)
  po = (lane == 2 * sub + 1).astype(jnp.float32)
  dn = (((1,), (0,)), ((), ()))

  def interleave(ev, od):
    evq = ev.reshape(_OR, _D)
    odq = od.reshape(_OR, _D)
    return lax.dot_general(evq, pe, dn) + lax.dot_general(odq, po, dn)

  dt = interleave(dte, dto)
  ds = interleave(dse, dso)

  t_out = jnp.maximum(dt + c_t, 0.0)
  s_out = ds + c_s
  w_pop = pop_ref[...] * wa
  w_time = t_out * wb
  w_side = s_out * wc
  opop_ref[...] = w_pop
  otime_ref[...] = w_time
  oside_ref[...] = w_side
  oout_ref[...] = w_pop + w_time + w_side


def kernel(pop_history, item, time, release_time, category, store,
           item_table, time_table, cat_table, store_table,
           gamma_time, beta_time, W_time, b_time,
           gamma_side, beta_side, W_side, b_side, attn_w):
  i32 = jnp.int32
  z7 = jnp.zeros((7, _D), jnp.float32)
  small_tab = jnp.concatenate([time_table, z7, cat_table, z7, store_table],
                              axis=0)
  time_e, rel_e, cat_e, store_e = _sc_gather_small(
      time.astype(i32), release_time.astype(i32),
      category.astype(i32) + _CAT_OFF, store.astype(i32) + _STORE_OFF,
      small_tab)
  item_e = _sc_gather_item(item.astype(i32), _item_repack(item_table.T))
  packed = [x.reshape(_B // 2, 128)
            for x in (item_e, time_e, rel_e, cat_e, store_e)]

  tf = time.astype(jnp.float32).reshape(_NPB, 1, _CB)
  pop3 = pl.pallas_call(
      _pop_body,
      grid=(_NPB,),
      in_specs=[
          pl.BlockSpec((_T, _CB), lambda b: (0, b)),
          pl.BlockSpec((1, 1, _CB), lambda b: (b, 0, 0)),
      ],
      out_specs=pl.BlockSpec((1, 1, _CB), lambda b: (b, 0, 0)),
      out_shape=jax.ShapeDtypeStruct((_NPB, 1, _CB), jnp.float32),
  )(pop_history.T, tf)
  pop_pk = pop3.reshape(128, 128)

  blk = lambda b: (b, 0)
  full = lambda b: (0, 0)
  espec = pl.BlockSpec((_PR, 128), blk)
  stats = pl.pallas_call(
      _stats_body,
      grid=(_NB,),
      in_specs=[espec] * 5,
      out_specs=pl.BlockSpec((16, _D), full),
      out_shape=jax.ShapeDtypeStruct((16, _D), jnp.float32),
      scratch_shapes=[pltpu.VMEM((16, _D), jnp.float32)],
  )(*packed)

  gt = gamma_time.reshape(4, _D)
  bt = beta_time.reshape(4, _D)
  wt = W_time.reshape(4, _D)
  bt0 = b_time.reshape(1, 1)
  gs = gamma_side.reshape(2, _D)
  bs = beta_side.reshape(2, _D)
  ws = W_side.reshape(2, _D)
  bs0 = b_side.reshape(1, 1)
  aw = attn_w.reshape(1, 3)

  out_spec = pl.BlockSpec((_OR, 128), blk)
  w_pop, w_time, w_side, output = pl.pallas_call(
      _final_body,
      grid=(_NB,),
      in_specs=[
          pl.BlockSpec((16, _D), full),
          pl.BlockSpec((_OR, 128), blk),
          espec, espec, espec, espec, espec,
          pl.BlockSpec((4, _D), full),
          pl.BlockSpec((4, _D), full),
          pl.BlockSpec((4, _D), full),
          pl.BlockSpec((1, 1), full),
          pl.BlockSpec((2, _D), full),
          pl.BlockSpec((2, _D), full),
          pl.BlockSpec((2, _D), full),
          pl.BlockSpec((1, 1), full),
          pl.BlockSpec((1, 3), full),
      ],
      out_specs=[out_spec] * 4,
      out_shape=[jax.ShapeDtypeStruct((128, 128), jnp.float32)] * 4,
  )(stats, pop_pk, *packed,
    gt, bt, wt, bt0, gs, bs, ws, bs0, aw)

  return tuple(o.reshape(_B, 1) for o in (w_pop, w_time, w_side, output))
